# baseline (device time: 232266 ns/iter reference)
import jax
import jax.numpy as jnp
from jax import lax
from jax.experimental import pallas as pl
from jax.experimental.pallas import tpu as pltpu

N_DEV = 4


def _ring_allreduce(partial):
    n, d = partial.shape

    def body(x_ref, out_ref, comm_ref, send_sems, recv_sems):
        my = lax.axis_index("i")
        left = lax.rem(my + N_DEV - 1, N_DEV)
        right = lax.rem(my + 1, N_DEV)

        barrier_sem = pltpu.get_barrier_semaphore()
        for nbr in (left, right):
            pl.semaphore_signal(
                barrier_sem, inc=1,
                device_id=(nbr,), device_id_type=pl.DeviceIdType.MESH,
            )
        pl.semaphore_wait(barrier_sem, 2)

        out_ref[...] = x_ref[...]
        comm_ref[0] = x_ref[...]

        for h in range(N_DEV - 1):
            rdma = pltpu.make_async_remote_copy(
                src_ref=comm_ref.at[h],
                dst_ref=comm_ref.at[h + 1],
                send_sem=send_sems.at[h],
                recv_sem=recv_sems.at[h],
                device_id=(right,),
                device_id_type=pl.DeviceIdType.MESH,
            )
            rdma.start()
            rdma.wait()
            out_ref[...] += comm_ref[h + 1]

    return pl.pallas_call(
        body,
        out_shape=jax.ShapeDtypeStruct((n, d), partial.dtype),
        in_specs=[pl.BlockSpec(memory_space=pltpu.VMEM)],
        out_specs=pl.BlockSpec(memory_space=pltpu.VMEM),
        scratch_shapes=[
            pltpu.VMEM((N_DEV, n, d), partial.dtype),
            pltpu.SemaphoreType.DMA((N_DEV - 1,)),
            pltpu.SemaphoreType.DMA((N_DEV - 1,)),
        ],
        compiler_params=pltpu.CompilerParams(collective_id=0),
    )(partial)


def kernel(table, idx):
    v_per, d = table.shape
    my = lax.axis_index("i")
    lidx = idx.astype(jnp.int32) - my * v_per
    owned = (lidx >= 0) & (lidx < v_per)
    rows = table[jnp.clip(lidx, 0, v_per - 1)]
    partial = jnp.where(owned[:, None], rows, 0).astype(jnp.bfloat16)
    return _ring_allreduce(partial)


# device time: 167878 ns/iter; 1.3835x vs baseline; 1.3835x over previous
import jax
import jax.numpy as jnp
from jax import lax
from jax.experimental import pallas as pl
from jax.experimental.pallas import tpu as pltpu

N_DEV = 4


def _ring_allreduce(partial):
    n, d = partial.shape
    s = n // N_DEV

    def body(x_ref, out_ref, rs_stage, rs_send, rs_recv, ag_send, ag_recv):
        my = lax.axis_index("i")
        left = lax.rem(my + N_DEV - 1, N_DEV)
        right = lax.rem(my + 1, N_DEV)

        barrier_sem = pltpu.get_barrier_semaphore()
        for nbr in (left, right):
            pl.semaphore_signal(
                barrier_sem, inc=1,
                device_id=(nbr,), device_id_type=pl.DeviceIdType.MESH,
            )
        pl.semaphore_wait(barrier_sem, 2)

        out_ref[...] = x_ref[...]

        def chunk(ref, c):
            return ref.at[pl.ds(c * s, s), :]

        for h in range(N_DEV - 1):
            send_c = lax.rem(my - h + N_DEV, N_DEV)
            recv_c = lax.rem(my - h - 1 + N_DEV, N_DEV)
            rdma = pltpu.make_async_remote_copy(
                src_ref=chunk(out_ref, send_c),
                dst_ref=rs_stage.at[h],
                send_sem=rs_send.at[h],
                recv_sem=rs_recv.at[h],
                device_id=(right,),
                device_id_type=pl.DeviceIdType.MESH,
            )
            rdma.start()
            rdma.wait()
            chunk(out_ref, recv_c)[...] += rs_stage[h]

        for h in range(N_DEV - 1):
            send_c = lax.rem(my + 1 - h + N_DEV, N_DEV)
            rdma = pltpu.make_async_remote_copy(
                src_ref=chunk(out_ref, send_c),
                dst_ref=chunk(out_ref, send_c),
                send_sem=ag_send.at[h],
                recv_sem=ag_recv.at[h],
                device_id=(right,),
                device_id_type=pl.DeviceIdType.MESH,
            )
            rdma.start()
            rdma.wait()

    return pl.pallas_call(
        body,
        out_shape=jax.ShapeDtypeStruct((n, d), partial.dtype),
        in_specs=[pl.BlockSpec(memory_space=pltpu.VMEM)],
        out_specs=pl.BlockSpec(memory_space=pltpu.VMEM),
        scratch_shapes=[
            pltpu.VMEM((N_DEV - 1, s, d), partial.dtype),
            pltpu.SemaphoreType.DMA((N_DEV - 1,)),
            pltpu.SemaphoreType.DMA((N_DEV - 1,)),
            pltpu.SemaphoreType.DMA((N_DEV - 1,)),
            pltpu.SemaphoreType.DMA((N_DEV - 1,)),
        ],
        compiler_params=pltpu.CompilerParams(collective_id=0),
    )(partial)


def kernel(table, idx):
    v_per, d = table.shape
    my = lax.axis_index("i")
    lidx = idx.astype(jnp.int32) - my * v_per
    owned = (lidx >= 0) & (lidx < v_per)
    rows = table[jnp.clip(lidx, 0, v_per - 1)]
    partial = jnp.where(owned[:, None], rows, 0).astype(jnp.bfloat16)
    return _ring_allreduce(partial)


# device time: 144299 ns/iter; 1.6096x vs baseline; 1.1634x over previous
import jax
import jax.numpy as jnp
from jax import lax
from jax.experimental import pallas as pl
from jax.experimental.pallas import tpu as pltpu

N_DEV = 4
UNROLL = 8


def _fused(lidx, mask, table):
    n = lidx.shape[0]
    v_per, d = table.shape
    s = n // N_DEV

    def body(lidx_ref, mask_ref, table_ref, out_ref,
             gath_ref, rs_stage, rs_send, rs_recv, ag_send, ag_recv, gsem):
        my = lax.axis_index("i")
        left = lax.rem(my + N_DEV - 1, N_DEV)
        right = lax.rem(my + 1, N_DEV)

        def row_dma(j):
            return pltpu.make_async_copy(
                table_ref.at[pl.ds(lidx_ref[j], 1), :],
                gath_ref.at[pl.ds(j, 1), :],
                gsem,
            )

        def issue(i, c):
            for u in range(UNROLL):
                row_dma(i * UNROLL + u).start()
            return c
        lax.fori_loop(0, n // UNROLL, issue, 0)

        barrier_sem = pltpu.get_barrier_semaphore()
        for nbr in (left, right):
            pl.semaphore_signal(
                barrier_sem, inc=1,
                device_id=(nbr,), device_id_type=pl.DeviceIdType.MESH,
            )
        pl.semaphore_wait(barrier_sem, 2)

        def drain(i, c):
            for u in range(UNROLL):
                row_dma(i * UNROLL + u).wait()
            return c
        lax.fori_loop(0, n // UNROLL, drain, 0)

        out_ref[...] = (gath_ref[...] * mask_ref[...]).astype(out_ref.dtype)

        def chunk(ref, c):
            return ref.at[pl.ds(c * s, s), :]

        for h in range(N_DEV - 1):
            send_c = lax.rem(my - h + N_DEV, N_DEV)
            recv_c = lax.rem(my - h - 1 + N_DEV, N_DEV)
            rdma = pltpu.make_async_remote_copy(
                src_ref=chunk(out_ref, send_c),
                dst_ref=rs_stage.at[h],
                send_sem=rs_send.at[h],
                recv_sem=rs_recv.at[h],
                device_id=(right,),
                device_id_type=pl.DeviceIdType.MESH,
            )
            rdma.start()
            rdma.wait()
            chunk(out_ref, recv_c)[...] += rs_stage[h]

        for h in range(N_DEV - 1):
            send_c = lax.rem(my + 1 - h + N_DEV, N_DEV)
            rdma = pltpu.make_async_remote_copy(
                src_ref=chunk(out_ref, send_c),
                dst_ref=chunk(out_ref, send_c),
                send_sem=ag_send.at[h],
                recv_sem=ag_recv.at[h],
                device_id=(right,),
                device_id_type=pl.DeviceIdType.MESH,
            )
            rdma.start()
            rdma.wait()

    return pl.pallas_call(
        body,
        out_shape=jax.ShapeDtypeStruct((n, d), jnp.bfloat16),
        in_specs=[
            pl.BlockSpec(memory_space=pltpu.SMEM),
            pl.BlockSpec(memory_space=pltpu.VMEM),
            pl.BlockSpec(memory_space=pltpu.HBM),
        ],
        out_specs=pl.BlockSpec(memory_space=pltpu.VMEM),
        scratch_shapes=[
            pltpu.VMEM((n, d), jnp.float32),
            pltpu.VMEM((N_DEV - 1, s, d), jnp.bfloat16),
            pltpu.SemaphoreType.DMA((N_DEV - 1,)),
            pltpu.SemaphoreType.DMA((N_DEV - 1,)),
            pltpu.SemaphoreType.DMA((N_DEV - 1,)),
            pltpu.SemaphoreType.DMA((N_DEV - 1,)),
            pltpu.SemaphoreType.DMA(()),
        ],
        compiler_params=pltpu.CompilerParams(collective_id=0),
    )(lidx, mask, table)


def kernel(table, idx):
    v_per, _ = table.shape
    my = lax.axis_index("i")
    lidx = idx.astype(jnp.int32) - my * v_per
    owned = (lidx >= 0) & (lidx < v_per)
    mask = owned.astype(jnp.float32)[:, None]
    return _fused(jnp.clip(lidx, 0, v_per - 1), mask, table)


# device time: 92397 ns/iter; 2.5138x vs baseline; 1.5617x over previous
import jax
import jax.numpy as jnp
from jax import lax
from jax.experimental import pallas as pl
from jax.experimental.pallas import tpu as pltpu

N_DEV = 4
U = 8


def _fused(lidx, mask, table):
    n = lidx.shape[0]
    v_per, d = table.shape
    half = n // 2
    s = half // N_DEV

    def body(lidx_ref, mask_ref, table_ref, out_ref,
             gath_ref, stage_cw, stage_ccw,
             rs_send, rs_recv, ag_send, ag_recv, gsem):
        my = lax.axis_index("i")
        left = lax.rem(my + N_DEV - 1, N_DEV)
        right = lax.rem(my + 1, N_DEV)

        def rmod(v):
            return lax.rem(v + 2 * N_DEV, N_DEV)

        def r_off(c):
            return c * s

        def l_off(c):
            return half + c * s

        def row_dma(j, sem):
            return pltpu.make_async_copy(
                table_ref.at[pl.ds(lidx_ref[j], 1), :],
                gath_ref.at[pl.ds(j, 1), :],
                sem,
            )

        def issue_rows(base, sem):
            def f(i, c):
                for u in range(U):
                    row_dma(base + i * U + u, sem).start()
                return c
            lax.fori_loop(0, s // U, f, 0)

        def drain_rows(base, sem):
            def f(i, c):
                for u in range(U):
                    row_dma(base + i * U + u, sem).wait()
                return c
            lax.fori_loop(0, s // U, f, 0)

        def stage_offs(k):
            return r_off(rmod(my - k)), l_off(rmod(my + k))

        def issue_stage(k):
            ro, lo = stage_offs(k)
            issue_rows(ro, gsem.at[k])
            issue_rows(lo, gsem.at[k])

        def finish_stage(k):
            ro, lo = stage_offs(k)
            drain_rows(ro, gsem.at[k])
            drain_rows(lo, gsem.at[k])
            for off in (ro, lo):
                out_ref[pl.ds(off, s), :] = (
                    gath_ref[pl.ds(off, s), :] * mask_ref[pl.ds(off, s), :]
                ).astype(out_ref.dtype)

        issue_stage(0)

        barrier_sem = pltpu.get_barrier_semaphore()
        for nbr in (left, right):
            pl.semaphore_signal(
                barrier_sem, inc=1,
                device_id=(nbr,), device_id_type=pl.DeviceIdType.MESH,
            )
        pl.semaphore_wait(barrier_sem, 2)

        finish_stage(0)

        for h in range(N_DEV - 1):
            cw = pltpu.make_async_remote_copy(
                src_ref=out_ref.at[pl.ds(r_off(rmod(my - h)), s), :],
                dst_ref=stage_cw.at[h],
                send_sem=rs_send.at[0, h],
                recv_sem=rs_recv.at[0, h],
                device_id=(right,),
                device_id_type=pl.DeviceIdType.MESH,
            )
            ccw = pltpu.make_async_remote_copy(
                src_ref=out_ref.at[pl.ds(l_off(rmod(my + h)), s), :],
                dst_ref=stage_ccw.at[h],
                send_sem=rs_send.at[1, h],
                recv_sem=rs_recv.at[1, h],
                device_id=(left,),
                device_id_type=pl.DeviceIdType.MESH,
            )
            cw.start()
            ccw.start()
            issue_stage(h + 1)
            finish_stage(h + 1)
            cw.wait()
            out_ref[pl.ds(r_off(rmod(my - h - 1)), s), :] += stage_cw[h]
            ccw.wait()
            out_ref[pl.ds(l_off(rmod(my + h + 1)), s), :] += stage_ccw[h]

        for h in range(N_DEV - 1):
            rc = r_off(rmod(my + 1 - h))
            lc = l_off(rmod(my - 1 + h))
            cw = pltpu.make_async_remote_copy(
                src_ref=out_ref.at[pl.ds(rc, s), :],
                dst_ref=out_ref.at[pl.ds(rc, s), :],
                send_sem=ag_send.at[0, h],
                recv_sem=ag_recv.at[0, h],
                device_id=(right,),
                device_id_type=pl.DeviceIdType.MESH,
            )
            ccw = pltpu.make_async_remote_copy(
                src_ref=out_ref.at[pl.ds(lc, s), :],
                dst_ref=out_ref.at[pl.ds(lc, s), :],
                send_sem=ag_send.at[1, h],
                recv_sem=ag_recv.at[1, h],
                device_id=(left,),
                device_id_type=pl.DeviceIdType.MESH,
            )
            cw.start()
            ccw.start()
            cw.wait()
            ccw.wait()

    return pl.pallas_call(
        body,
        out_shape=jax.ShapeDtypeStruct((n, d), jnp.bfloat16),
        in_specs=[
            pl.BlockSpec(memory_space=pltpu.SMEM),
            pl.BlockSpec(memory_space=pltpu.VMEM),
            pl.BlockSpec(memory_space=pltpu.HBM),
        ],
        out_specs=pl.BlockSpec(memory_space=pltpu.VMEM),
        scratch_shapes=[
            pltpu.VMEM((n, d), jnp.float32),
            pltpu.VMEM((N_DEV - 1, s, d), jnp.bfloat16),
            pltpu.VMEM((N_DEV - 1, s, d), jnp.bfloat16),
            pltpu.SemaphoreType.DMA((2, N_DEV - 1)),
            pltpu.SemaphoreType.DMA((2, N_DEV - 1)),
            pltpu.SemaphoreType.DMA((2, N_DEV - 1)),
            pltpu.SemaphoreType.DMA((2, N_DEV - 1)),
            pltpu.SemaphoreType.DMA((N_DEV,)),
        ],
        compiler_params=pltpu.CompilerParams(collective_id=0),
    )(lidx, mask, table)


def kernel(table, idx):
    v_per, _ = table.shape
    my = lax.axis_index("i")
    lidx = idx.astype(jnp.int32) - my * v_per
    owned = (lidx >= 0) & (lidx < v_per)
    mask = owned.astype(jnp.float32)[:, None]
    return _fused(jnp.clip(lidx, 0, v_per - 1), mask, table)


# device time: 71416 ns/iter; 3.2523x vs baseline; 1.2938x over previous
import jax
import jax.numpy as jnp
from jax import lax
from jax.experimental import pallas as pl
from jax.experimental.pallas import tpu as pltpu

N_DEV = 4
U = 8


def _fused(lidx, owned, ccounts, mask, table):
    n = lidx.shape[0]
    v_per, d = table.shape
    half = n // 2
    s = half // N_DEV

    def body(lidx_ref, owned_ref, ccount_ref, mask_ref, table_ref, out_ref,
             gath_ref, stage_cw, stage_ccw,
             rs_send, rs_recv, ag_send, ag_recv, gsem):
        my = lax.axis_index("i")
        left = lax.rem(my + N_DEV - 1, N_DEV)
        right = lax.rem(my + 1, N_DEV)

        def rmod(v):
            return lax.rem(v + 2 * N_DEV, N_DEV)

        def r_off(c):
            return c * s

        def l_off(c):
            return half + c * s

        def row_dma(j, sem):
            return pltpu.make_async_copy(
                table_ref.at[pl.ds(lidx_ref[j], 1), :],
                gath_ref.at[pl.ds(j, 1), :],
                sem,
            )

        def issue_rows(base, sem):
            def f(i, c):
                for u in range(U):
                    j = base + i * U + u

                    @pl.when(owned_ref[j] != 0)
                    def _():
                        row_dma(j, sem).start()
                return c
            lax.fori_loop(0, s // U, f, 0)

        def drain_rows(c8, sem):
            def f(i, c):
                row_dma(0, sem).wait()
                return c
            lax.fori_loop(0, ccount_ref[c8], f, 0)

        def stage_chunks(k):
            return rmod(my - k), N_DEV + rmod(my + k)

        def issue_stage(k):
            cr, cl = stage_chunks(k)
            issue_rows(cr * s, gsem.at[k])
            issue_rows(cl * s, gsem.at[k])

        def finish_stage(k):
            cr, cl = stage_chunks(k)
            drain_rows(cr, gsem.at[k])
            drain_rows(cl, gsem.at[k])
            for c8 in (cr, cl):
                off = c8 * s
                out_ref[pl.ds(off, s), :] = (
                    gath_ref[pl.ds(off, s), :] * mask_ref[pl.ds(off, s), :]
                ).astype(out_ref.dtype)

        issue_stage(0)

        barrier_sem = pltpu.get_barrier_semaphore()
        for nbr in (left, right):
            pl.semaphore_signal(
                barrier_sem, inc=1,
                device_id=(nbr,), device_id_type=pl.DeviceIdType.MESH,
            )
        pl.semaphore_wait(barrier_sem, 2)

        finish_stage(0)

        for h in range(N_DEV - 1):
            cw = pltpu.make_async_remote_copy(
                src_ref=out_ref.at[pl.ds(r_off(rmod(my - h)), s), :],
                dst_ref=stage_cw.at[h],
                send_sem=rs_send.at[0, h],
                recv_sem=rs_recv.at[0, h],
                device_id=(right,),
                device_id_type=pl.DeviceIdType.MESH,
            )
            ccw = pltpu.make_async_remote_copy(
                src_ref=out_ref.at[pl.ds(l_off(rmod(my + h)), s), :],
                dst_ref=stage_ccw.at[h],
                send_sem=rs_send.at[1, h],
                recv_sem=rs_recv.at[1, h],
                device_id=(left,),
                device_id_type=pl.DeviceIdType.MESH,
            )
            cw.start()
            ccw.start()
            issue_stage(h + 1)
            finish_stage(h + 1)
            cw.wait()
            out_ref[pl.ds(r_off(rmod(my - h - 1)), s), :] += stage_cw[h]
            ccw.wait()
            out_ref[pl.ds(l_off(rmod(my + h + 1)), s), :] += stage_ccw[h]

        for h in range(N_DEV - 1):
            rc = r_off(rmod(my + 1 - h))
            lc = l_off(rmod(my - 1 + h))
            cw = pltpu.make_async_remote_copy(
                src_ref=out_ref.at[pl.ds(rc, s), :],
                dst_ref=out_ref.at[pl.ds(rc, s), :],
                send_sem=ag_send.at[0, h],
                recv_sem=ag_recv.at[0, h],
                device_id=(right,),
                device_id_type=pl.DeviceIdType.MESH,
            )
            ccw = pltpu.make_async_remote_copy(
                src_ref=out_ref.at[pl.ds(lc, s), :],
                dst_ref=out_ref.at[pl.ds(lc, s), :],
                send_sem=ag_send.at[1, h],
                recv_sem=ag_recv.at[1, h],
                device_id=(left,),
                device_id_type=pl.DeviceIdType.MESH,
            )
            cw.start()
            ccw.start()
            cw.wait()
            ccw.wait()

    return pl.pallas_call(
        body,
        out_shape=jax.ShapeDtypeStruct((n, d), jnp.bfloat16),
        in_specs=[
            pl.BlockSpec(memory_space=pltpu.SMEM),
            pl.BlockSpec(memory_space=pltpu.SMEM),
            pl.BlockSpec(memory_space=pltpu.SMEM),
            pl.BlockSpec(memory_space=pltpu.VMEM),
            pl.BlockSpec(memory_space=pltpu.HBM),
        ],
        out_specs=pl.BlockSpec(memory_space=pltpu.VMEM),
        scratch_shapes=[
            pltpu.VMEM((n, d), jnp.float32),
            pltpu.VMEM((N_DEV - 1, s, d), jnp.bfloat16),
            pltpu.VMEM((N_DEV - 1, s, d), jnp.bfloat16),
            pltpu.SemaphoreType.DMA((2, N_DEV - 1)),
            pltpu.SemaphoreType.DMA((2, N_DEV - 1)),
            pltpu.SemaphoreType.DMA((2, N_DEV - 1)),
            pltpu.SemaphoreType.DMA((2, N_DEV - 1)),
            pltpu.SemaphoreType.DMA((N_DEV,)),
        ],
        compiler_params=pltpu.CompilerParams(collective_id=0),
    )(lidx, owned, ccounts, mask, table)


def kernel(table, idx):
    v_per, _ = table.shape
    my = lax.axis_index("i")
    lidx = idx.astype(jnp.int32) - my * v_per
    owned = ((lidx >= 0) & (lidx < v_per)).astype(jnp.int32)
    mask = owned.astype(jnp.float32)[:, None]
    ccounts = owned.reshape(2 * N_DEV, -1).sum(axis=1, dtype=jnp.int32)
    return _fused(jnp.clip(lidx, 0, v_per - 1), owned, ccounts, mask, table)


# device time: 51934 ns/iter; 4.4723x vs baseline; 1.3751x over previous
import jax
import jax.numpy as jnp
from jax import lax
from jax.experimental import pallas as pl
from jax.experimental.pallas import tpu as pltpu

N_DEV = 4
N_SUB = 2

CW, CCW = 0, 1


def _fused(lidx, packed, ccounts, mask, table):
    n = lidx.shape[0]
    v_per, d = table.shape
    half = n // 2
    s = half // N_DEV
    s2 = s // N_SUB

    def body(lidx_ref, packed_ref, ccount_ref, mask_ref, table_ref, out_ref,
             gath_ref, stage_cw, stage_ccw, send_sems, recv_sems, ag_recv,
             gsem):
        my = lax.axis_index("i")
        left = lax.rem(my + N_DEV - 1, N_DEV)
        right = lax.rem(my + 1, N_DEV)

        def rmod(v):
            return lax.rem(v + 2 * N_DEV, N_DEV)

        def row_dma(j, sem):
            return pltpu.make_async_copy(
                table_ref.at[pl.ds(lidx_ref[j], 1), :],
                gath_ref.at[pl.ds(j, 1), :],
                sem,
            )

        def stage_chunks(k):
            return rmod(my - k), N_DEV + rmod(my + k)

        def issue_stage(k):
            for c8 in stage_chunks(k):
                base = c8 * s

                def f(i, c):
                    row_dma(packed_ref[base + i], gsem.at[k]).start()
                    return c
                lax.fori_loop(0, ccount_ref[c8], f, 0)

        def finish_stage(k):
            cr, cl = stage_chunks(k)
            for c8 in (cr, cl):
                def f(i, c):
                    row_dma(0, gsem.at[k]).wait()
                    return c
                lax.fori_loop(0, ccount_ref[c8], f, 0)
            for c8 in (cr, cl):
                off = c8 * s
                out_ref[pl.ds(off, s), :] = (
                    gath_ref[pl.ds(off, s), :] * mask_ref[pl.ds(off, s), :]
                ).astype(out_ref.dtype)

        def rs_chunk(dirn, h):
            if dirn == CW:
                return rmod(my - h)
            return N_DEV + rmod(my + h)

        def rs_acc_chunk(dirn, h):
            if dirn == CW:
                return rmod(my - h - 1)
            return N_DEV + rmod(my + h + 1)

        def ag_chunk(dirn, h):
            if dirn == CW:
                return rmod(my + 1 - h)
            return N_DEV + rmod(my - 1 + h)

        def peer(dirn):
            return right if dirn == CW else left

        def sub(ref, c8, k):
            return ref.at[pl.ds(c8 * s + k * s2, s2), :]

        def rs_desc(dirn, h, k):
            stage = stage_cw if dirn == CW else stage_ccw
            return pltpu.make_async_remote_copy(
                src_ref=sub(out_ref, rs_chunk(dirn, h), k),
                dst_ref=stage.at[h, pl.ds(k * s2, s2), :],
                send_sem=send_sems.at[dirn, h, k],
                recv_sem=recv_sems.at[dirn, h, k],
                device_id=(peer(dirn),),
                device_id_type=pl.DeviceIdType.MESH,
            )

        def ag_desc(dirn, h, k):
            region = sub(out_ref, ag_chunk(dirn, h), k)
            return pltpu.make_async_remote_copy(
                src_ref=region,
                dst_ref=region,
                send_sem=send_sems.at[dirn, h, k],
                recv_sem=ag_recv.at[dirn, h, k],
                device_id=(peer(dirn),),
                device_id_type=pl.DeviceIdType.MESH,
            )

        def rs_add(dirn, h, k):
            stage = stage_cw if dirn == CW else stage_ccw
            c8 = rs_acc_chunk(dirn, h)
            out_ref[pl.ds(c8 * s + k * s2, s2), :] += (
                stage[h, pl.ds(k * s2, s2), :]
            )

        issue_stage(0)

        barrier_sem = pltpu.get_barrier_semaphore()
        for nbr in (left, right):
            pl.semaphore_signal(
                barrier_sem, inc=1,
                device_id=(nbr,), device_id_type=pl.DeviceIdType.MESH,
            )
        pl.semaphore_wait(barrier_sem, 2)

        finish_stage(0)

        ag_descs = {}
        for dirn in (CW, CCW):
            for k in range(N_SUB):
                rs_desc(dirn, 0, k).start()

        for h in range(N_DEV - 1):
            issue_stage(h + 1)
            finish_stage(h + 1)
            for k in range(N_SUB):
                for dirn in (CW, CCW):
                    rs_desc(dirn, h, k).wait()
                    rs_add(dirn, h, k)
                    if h + 1 < N_DEV - 1:
                        rs_desc(dirn, h + 1, k).start()
                    else:
                        ag = ag_desc(dirn, 0, k)
                        ag_descs[dirn, 0, k] = ag
                        ag.start()

        for h in range(1, N_DEV - 1):
            for k in range(N_SUB):
                for dirn in (CW, CCW):
                    ag_descs[dirn, h - 1, k].wait_recv()
                    ag = ag_desc(dirn, h, k)
                    ag_descs[dirn, h, k] = ag
                    ag.start()
        for k in range(N_SUB):
            for dirn in (CW, CCW):
                ag_descs[dirn, N_DEV - 2, k].wait_recv()
        for (dirn, h, k), ag in ag_descs.items():
            ag.wait_send()

    return pl.pallas_call(
        body,
        out_shape=jax.ShapeDtypeStruct((n, d), jnp.bfloat16),
        in_specs=[
            pl.BlockSpec(memory_space=pltpu.SMEM),
            pl.BlockSpec(memory_space=pltpu.SMEM),
            pl.BlockSpec(memory_space=pltpu.SMEM),
            pl.BlockSpec(memory_space=pltpu.VMEM),
            pl.BlockSpec(memory_space=pltpu.HBM),
        ],
        out_specs=pl.BlockSpec(memory_space=pltpu.VMEM),
        scratch_shapes=[
            pltpu.VMEM((n, d), jnp.float32),
            pltpu.VMEM((N_DEV - 1, s, d), jnp.bfloat16),
            pltpu.VMEM((N_DEV - 1, s, d), jnp.bfloat16),
            pltpu.SemaphoreType.DMA((2, N_DEV - 1, N_SUB)),
            pltpu.SemaphoreType.DMA((2, N_DEV - 1, N_SUB)),
            pltpu.SemaphoreType.DMA((2, N_DEV - 1, N_SUB)),
            pltpu.SemaphoreType.DMA((N_DEV,)),
        ],
        compiler_params=pltpu.CompilerParams(collective_id=0),
    )(lidx, packed, ccounts, mask, table)


def kernel(table, idx):
    v_per, _ = table.shape
    n = idx.shape[0]
    my = lax.axis_index("i")
    lidx = idx.astype(jnp.int32) - my * v_per
    owned = ((lidx >= 0) & (lidx < v_per)).astype(jnp.int32)
    mask = owned.astype(jnp.float32)[:, None]
    s = n // (2 * N_DEV)
    owned2d = owned.reshape(2 * N_DEV, s)
    ccounts = owned2d.sum(axis=1, dtype=jnp.int32)
    order = jnp.argsort(1 - owned2d, axis=1, stable=True)
    packed = (
        jnp.arange(2 * N_DEV, dtype=jnp.int32)[:, None] * s
        + order.astype(jnp.int32)
    ).reshape(-1)
    return _fused(jnp.clip(lidx, 0, v_per - 1), packed, ccounts, mask, table)
